# Initial kernel scaffold; baseline (speedup 1.0000x reference)
import jax
import jax.numpy as jnp
from jax.experimental import pallas as pl


def kernel(node_feat, node_pos, node_vel, edge_index, edge_attr, msg_W1, msg_b1, msg_W2, msg_b2, pos_W1, pos_b1, pos_W2, pos_b2, node_W1, node_b1, node_W2, node_b2, vel_W1, vel_b1, vel_W2, vel_b2):
    raise NotImplementedError("write your pallas kernel here")


# speedup vs baseline: 3.7990x; 3.7990x over previous
import functools

import jax
import jax.numpy as jnp
from jax import lax
from jax.experimental import pallas as pl
from jax.experimental.pallas import tpu as pltpu
from jax.experimental.pallas import tpu_sc as plsc

N = 10000
E = 320000
H = 128
NC, NS = 2, 16
NW = NC * NS
EPW = E // NW
CH = 128
NFULL = EPW // CH
REM = EPW - NFULL * CH
NCHUNK = NFULL + 1
NCHUNK_PAD = 80
NPAD = 10112
RPT = NPAD // NS


def _tables_body(nf_ref, wa_ref, wb_ref, ta_ref, tb_ref):
    nf = nf_ref[...]
    ta_ref[...] = jnp.dot(nf, wa_ref[...], preferred_element_type=jnp.float32)
    tb_ref[...] = jnp.dot(nf, wb_ref[...], preferred_element_type=jnp.float32)


_BT = 2528


def _tables_call(nf_p, wa, wb):
    return pl.pallas_call(
        _tables_body,
        grid=(NPAD // _BT,),
        in_specs=[
            pl.BlockSpec((_BT, H), lambda i: (i, 0)),
            pl.BlockSpec((H, H), lambda i: (0, 0)),
            pl.BlockSpec((H, H), lambda i: (0, 0)),
        ],
        out_specs=[
            pl.BlockSpec((_BT, H), lambda i: (i, 0)),
            pl.BlockSpec((_BT, H), lambda i: (i, 0)),
        ],
        out_shape=[
            jax.ShapeDtypeStruct((NPAD, H), jnp.float32),
            jax.ShapeDtypeStruct((NPAD, H), jnp.float32),
        ],
    )(nf_p, wa, wb)


@functools.cache
def _sc_mesh():
    return plsc.VectorSubcoreMesh(
        core_axis_name="c", subcore_axis_name="s", num_cores=NC, num_subcores=NS)


@functools.cache
def _gather_kernel_build():
    return pl.kernel(
        _gather_body,
        out_type=[
            jax.ShapeDtypeStruct((E, H), jnp.float32),
            jax.ShapeDtypeStruct((3 * E,), jnp.float32),
        ],
        mesh=_sc_mesh(),
        scratch_types=[
            pltpu.VMEM((NCHUNK_PAD, CH), jnp.int32),
            pltpu.VMEM((NCHUNK_PAD, CH), jnp.int32),
            pltpu.VMEM((CH, H), jnp.float32),
            pltpu.VMEM((CH, H), jnp.float32),
            pltpu.VMEM((CH, H), jnp.float32),
            pltpu.VMEM((CH,), jnp.float32),
            pltpu.VMEM((CH,), jnp.float32),
            pltpu.VMEM((CH,), jnp.float32),
            pltpu.VMEM((CH,), jnp.float32),
            pltpu.VMEM((CH,), jnp.float32),
            pltpu.VMEM((CH,), jnp.float32),
            pltpu.VMEM((3 * CH,), jnp.float32),
            pltpu.SemaphoreType.DMA,
            pltpu.SemaphoreType.DMA,
        ],
    )


def _gather_body(ta_hbm, tb_hbm, px_hbm, py_hbm, pz_hbm, ir_hbm, ic_hbm,
                 pre_hbm, aux_hbm,
                 ir_v, ic_v, abuf, bbuf, obuf,
                 xr, xc, yr, yc, zr, zc, xbuf, sem_a, sem_b):
    c = lax.axis_index("c")
    s = lax.axis_index("s")
    w = c * NS + s
    base = w * EPW
    pltpu.sync_copy(ir_hbm.at[pl.ds(w * NCHUNK_PAD, NCHUNK_PAD)], ir_v)
    pltpu.sync_copy(ic_hbm.at[pl.ds(w * NCHUNK_PAD, NCHUNK_PAD)], ic_v)

    def do_chunk(j, nrows):
        ca = pltpu.async_copy(ta_hbm.at[ir_v.at[j]], abuf, sem_a)
        cb = pltpu.async_copy(tb_hbm.at[ic_v.at[j]], bbuf, sem_b)
        pltpu.sync_copy(px_hbm.at[ir_v.at[j]], xr)
        pltpu.sync_copy(px_hbm.at[ic_v.at[j]], xc)
        pltpu.sync_copy(py_hbm.at[ir_v.at[j]], yr)
        pltpu.sync_copy(py_hbm.at[ic_v.at[j]], yc)
        pltpu.sync_copy(pz_hbm.at[ir_v.at[j]], zr)
        pltpu.sync_copy(pz_hbm.at[ic_v.at[j]], zc)
        ca.wait()
        cb.wait()

        @pl.loop(0, nrows)
        def _(i):
            for t in range(H // 16):
                sl = pl.ds(t * 16, 16)
                obuf[i, sl] = abuf[i, sl] + bbuf[i, sl]

        for g in range(nrows // 16):
            sl = pl.ds(g * 16, 16)
            xbuf[pl.ds(0 * CH + g * 16, 16)] = xr[sl] - xc[sl]
            xbuf[pl.ds(1 * CH + g * 16, 16)] = yr[sl] - yc[sl]
            xbuf[pl.ds(2 * CH + g * 16, 16)] = zr[sl] - zc[sl]

        pltpu.sync_copy(obuf.at[pl.ds(0, nrows)],
                        pre_hbm.at[pl.ds(base + j * CH, nrows)])
        for k in range(3):
            pltpu.sync_copy(xbuf.at[pl.ds(k * CH, nrows)],
                            aux_hbm.at[pl.ds(k * E + base + j * CH, nrows)])

    @pl.loop(0, NFULL)
    def _(j):
        do_chunk(j, CH)

    do_chunk(NFULL, REM)


def _edge_body(pre_ref, dp_ref, ea_ref, w1c_ref, w1d_ref, b1_ref, w2_ref,
               b2_ref, pw1_ref, pb1_ref, pw2_ref, pb2_ref,
               msg_ref, wp_ref):
    pre = pre_ref[...]
    dp = dp_ref[...]
    dist = jnp.sum(dp * dp, axis=0, keepdims=True)
    ec = jnp.dot(ea_ref[...], w1c_ref[...], preferred_element_type=jnp.float32)
    h = jax.nn.silu(pre + ec + dist.T * w1d_ref[...] + b1_ref[...])
    msg = jax.nn.silu(
        jnp.dot(h, w2_ref[...], preferred_element_type=jnp.float32) + b2_ref[...])
    p1 = jax.nn.silu(
        jnp.dot(msg, pw1_ref[...], preferred_element_type=jnp.float32) + pb1_ref[...])
    sc = jnp.dot(p1, pw2_ref[...], preferred_element_type=jnp.float32) + pb2_ref[...]
    msg_ref[...] = msg
    wp_ref[...] = dp * sc.T


_BE = 2560


def _edge_call(pre, dp, ea, w1c, w1d, b1, w2, b2, pw1, pb1, pw2, pb2):
    cmap = lambda i: (0, 0)
    return pl.pallas_call(
        _edge_body,
        grid=(E // _BE,),
        in_specs=[
            pl.BlockSpec((_BE, H), lambda i: (i, 0)),
            pl.BlockSpec((3, _BE), lambda i: (0, i)),
            pl.BlockSpec((_BE, 16), lambda i: (i, 0)),
            pl.BlockSpec((16, H), cmap),
            pl.BlockSpec((1, H), cmap),
            pl.BlockSpec((1, H), cmap),
            pl.BlockSpec((H, H), cmap),
            pl.BlockSpec((1, H), cmap),
            pl.BlockSpec((H, H), cmap),
            pl.BlockSpec((1, H), cmap),
            pl.BlockSpec((H, 1), cmap),
            pl.BlockSpec((1, 1), cmap),
        ],
        out_specs=[
            pl.BlockSpec((_BE, H), lambda i: (i, 0)),
            pl.BlockSpec((3, _BE), lambda i: (0, i)),
        ],
        out_shape=[
            jax.ShapeDtypeStruct((E, H), jnp.float32),
            jax.ShapeDtypeStruct((3, E), jnp.float32),
        ],
    )(pre, dp, ea, w1c, w1d, b1, w2, b2, pw1, pb1, pw2, pb2)


@functools.cache
def _scatter_kernel_build():
    return pl.kernel(
        _scatter_body,
        out_type=[
            jax.ShapeDtypeStruct((NC * NPAD, H), jnp.float32),
            jax.ShapeDtypeStruct((NC * 4 * NPAD,), jnp.float32),
        ],
        mesh=_sc_mesh(),
        scratch_types=[
            pltpu.VMEM((NCHUNK_PAD, CH), jnp.int32),
            pltpu.VMEM((CH, H), jnp.float32),
            pltpu.VMEM((CH,), jnp.float32),
            pltpu.VMEM((CH,), jnp.float32),
            pltpu.VMEM((CH,), jnp.float32),
            pltpu.VMEM((CH,), jnp.float32),
            pltpu.VMEM((640,), jnp.float32),
            pltpu.VMEM_SHARED((NPAD, H), jnp.float32),
            pltpu.VMEM_SHARED((NPAD,), jnp.float32),
            pltpu.VMEM_SHARED((NPAD,), jnp.float32),
            pltpu.VMEM_SHARED((NPAD,), jnp.float32),
            pltpu.VMEM_SHARED((NPAD,), jnp.float32),
        ],
    )


def _scatter_body(msg_hbm, wp_hbm, is_hbm, outm_hbm, outx_hbm,
                  is_v, bufm, bx, by, bz, ones, zbuf,
                  accm, accx, accy, accz, accc):
    c = lax.axis_index("c")
    s = lax.axis_index("s")
    w = c * NS + s
    base = w * EPW
    r0 = s * RPT
    nz = RPT // CH
    rz = RPT - nz * CH

    @pl.loop(0, CH)
    def _(i):
        for t in range(H // 16):
            bufm[i, pl.ds(t * 16, 16)] = jnp.zeros((16,), jnp.float32)

    @pl.loop(0, 40)
    def _(i):
        zbuf[pl.ds(i * 16, 16)] = jnp.zeros((16,), jnp.float32)

    @pl.loop(0, CH // 16)
    def _(i):
        ones[pl.ds(i * 16, 16)] = jnp.full((16,), 1.0, jnp.float32)

    @pl.loop(0, nz)
    def _(k):
        pltpu.sync_copy(bufm, accm.at[pl.ds(r0 + k * CH, CH)])

    pltpu.sync_copy(bufm.at[pl.ds(0, rz)], accm.at[pl.ds(r0 + nz * CH, rz)])
    for acc in (accx, accy, accz, accc):
        pltpu.sync_copy(zbuf.at[pl.ds(0, RPT)], acc.at[pl.ds(r0, RPT)])
    plsc.subcore_barrier()

    pltpu.sync_copy(is_hbm.at[pl.ds(w * NCHUNK_PAD, NCHUNK_PAD)], is_v)

    def do_chunk(j, nrows):
        pltpu.sync_copy(msg_hbm.at[pl.ds(base + j * CH, nrows)],
                        bufm.at[pl.ds(0, nrows)])
        pltpu.sync_copy(wp_hbm.at[pl.ds(0 * E + base + j * CH, nrows)],
                        bx.at[pl.ds(0, nrows)])
        pltpu.sync_copy(wp_hbm.at[pl.ds(1 * E + base + j * CH, nrows)],
                        by.at[pl.ds(0, nrows)])
        pltpu.sync_copy(wp_hbm.at[pl.ds(2 * E + base + j * CH, nrows)],
                        bz.at[pl.ds(0, nrows)])
        idx = is_v.at[j]
        pltpu.sync_copy(bufm, accm.at[idx], add=True)
        pltpu.sync_copy(bx, accx.at[idx], add=True)
        pltpu.sync_copy(by, accy.at[idx], add=True)
        pltpu.sync_copy(bz, accz.at[idx], add=True)
        pltpu.sync_copy(ones, accc.at[idx], add=True)

    @pl.loop(0, NFULL)
    def _(j):
        do_chunk(j, CH)

    do_chunk(NFULL, REM)
    plsc.subcore_barrier()

    @pl.loop(0, nz)
    def _(k):
        pltpu.sync_copy(accm.at[pl.ds(r0 + k * CH, CH)], bufm)
        pltpu.sync_copy(bufm, outm_hbm.at[pl.ds(c * NPAD + r0 + k * CH, CH)])

    pltpu.sync_copy(accm.at[pl.ds(r0 + nz * CH, rz)], bufm.at[pl.ds(0, rz)])
    pltpu.sync_copy(bufm.at[pl.ds(0, rz)],
                    outm_hbm.at[pl.ds(c * NPAD + nz * CH + r0, rz)])
    for k, acc in enumerate((accx, accy, accz, accc)):
        pltpu.sync_copy(acc.at[pl.ds(r0, RPT)], zbuf.at[pl.ds(0, RPT)])
        pltpu.sync_copy(zbuf.at[pl.ds(0, RPT)],
                        outx_hbm.at[pl.ds((c * 4 + k) * NPAD + r0, RPT)])


def _node_body(accm_ref, accx_ref, nf_ref, pos_ref, vel_ref,
               nw1a_ref, nw1b_ref, nb1_ref, nw2_ref, nb2_ref,
               vw1_ref, vb1_ref, vw2_ref, vb2_ref,
               feat_out_ref, pos_out_ref):
    aux = accx_ref[0] + accx_ref[1]
    cnt = jnp.maximum(aux[:, 3:4], 1.0)
    magg = (accm_ref[0] + accm_ref[1]) / cnt
    pagg = aux[:, :3] / cnt
    nf = nf_ref[...]
    h = jax.nn.silu(
        jnp.dot(nf, nw1a_ref[...], preferred_element_type=jnp.float32)
        + jnp.dot(magg, nw1b_ref[...], preferred_element_type=jnp.float32)
        + nb1_ref[...])
    feat_out_ref[...] = (
        jnp.dot(h, nw2_ref[...], preferred_element_type=jnp.float32) + nb2_ref[...])
    v = jax.nn.silu(
        jnp.dot(nf, vw1_ref[...], preferred_element_type=jnp.float32) + vb1_ref[...])
    vs = jnp.dot(v, vw2_ref[...], preferred_element_type=jnp.float32) + vb2_ref[...]
    pos_out_ref[...] = pos_ref[...] + pagg + vs * vel_ref[...]


_BN = 2000


def _node_call(accm, accx, nf, pos, vel, nw1a, nw1b, nb1, nw2, nb2,
               vw1, vb1, vw2, vb2):
    cmap = lambda i: (0, 0)
    return pl.pallas_call(
        _node_body,
        grid=(N // _BN,),
        in_specs=[
            pl.BlockSpec((2, _BN, H), lambda i: (0, i, 0)),
            pl.BlockSpec((2, _BN, 4), lambda i: (0, i, 0)),
            pl.BlockSpec((_BN, H), lambda i: (i, 0)),
            pl.BlockSpec((_BN, 3), lambda i: (i, 0)),
            pl.BlockSpec((_BN, 3), lambda i: (i, 0)),
            pl.BlockSpec((H, H), cmap),
            pl.BlockSpec((H, H), cmap),
            pl.BlockSpec((1, H), cmap),
            pl.BlockSpec((H, H), cmap),
            pl.BlockSpec((1, H), cmap),
            pl.BlockSpec((H, H), cmap),
            pl.BlockSpec((1, H), cmap),
            pl.BlockSpec((H, 1), cmap),
            pl.BlockSpec((1, 1), cmap),
        ],
        out_specs=[
            pl.BlockSpec((_BN, H), lambda i: (i, 0)),
            pl.BlockSpec((_BN, 3), lambda i: (i, 0)),
        ],
        out_shape=[
            jax.ShapeDtypeStruct((N, H), jnp.float32),
            jax.ShapeDtypeStruct((N, 3), jnp.float32),
        ],
    )(accm, accx, nf, pos, vel, nw1a, nw1b, nb1, nw2, nb2, vw1, vb1, vw2, vb2)


def _pad_idx(ix):
    ix = ix.reshape(NW, EPW)
    pad = jnp.full((NW, NCHUNK_PAD * CH - EPW), N, jnp.int32)
    return jnp.concatenate([ix, pad], axis=1).reshape(NW * NCHUNK_PAD, CH)


def kernel(node_feat, node_pos, node_vel, edge_index, edge_attr,
           msg_W1, msg_b1, msg_W2, msg_b2,
           pos_W1, pos_b1, pos_W2, pos_b2,
           node_W1, node_b1, node_W2, node_b2,
           vel_W1, vel_b1, vel_W2, vel_b2):
    ir = _pad_idx(edge_index[0])
    ic = _pad_idx(edge_index[1])
    nf_p = jnp.pad(node_feat, ((0, NPAD - N), (0, 0)))
    pos_p = jnp.pad(node_pos, ((0, NPAD - N), (0, 0)))
    px, py, pz = pos_p[:, 0], pos_p[:, 1], pos_p[:, 2]

    ta, tb = _tables_call(nf_p, msg_W1[:H], msg_W1[H:2 * H])
    pre, dp = _gather_kernel_build()(ta, tb, px, py, pz, ir, ic)
    msg, wp = _edge_call(pre, dp.reshape(3, E), edge_attr,
                         msg_W1[2 * H:2 * H + 16], msg_W1[2 * H + 16:],
                         msg_b1.reshape(1, H), msg_W2, msg_b2.reshape(1, H),
                         pos_W1, pos_b1.reshape(1, H), pos_W2,
                         pos_b2.reshape(1, 1))
    accm, accx = _scatter_kernel_build()(msg, wp.reshape(3 * E), ir)
    accm = accm.reshape(NC, NPAD, H)
    accx = jnp.transpose(accx.reshape(NC, 4, NPAD), (0, 2, 1))
    return _node_call(accm, accx, node_feat, node_pos, node_vel,
                      node_W1[:H], node_W1[H:], node_b1.reshape(1, H),
                      node_W2, node_b2.reshape(1, H),
                      vel_W1, vel_b1.reshape(1, H), vel_W2, vel_b2.reshape(1, 1))


# speedup vs baseline: 5.2350x; 1.3780x over previous
import functools

import jax
import jax.numpy as jnp
from jax import lax
from jax.experimental import pallas as pl
from jax.experimental.pallas import tpu as pltpu
from jax.experimental.pallas import tpu_sc as plsc

N = 10000
E = 320000
H = 128
NC, NS = 2, 16
NW = NC * NS
EPW = E // NW
CH = 128
NFULL = EPW // CH
REM = EPW - NFULL * CH
NCHUNK = NFULL + 1
NCHUNK_PAD = 80
NPAD = 10112
RPT = NPAD // NS
EPAD = E + CH - REM


def _tables_body(nf_ref, wa_ref, wb_ref, ta_ref, tb_ref):
    nf = nf_ref[...]
    ta_ref[...] = jnp.dot(nf, wa_ref[...], preferred_element_type=jnp.float32)
    tb_ref[...] = jnp.dot(nf, wb_ref[...], preferred_element_type=jnp.float32)


_BT = 2528


def _tables_call(nf_p, wa, wb):
    return pl.pallas_call(
        _tables_body,
        grid=(NPAD // _BT,),
        in_specs=[
            pl.BlockSpec((_BT, H), lambda i: (i, 0)),
            pl.BlockSpec((H, H), lambda i: (0, 0)),
            pl.BlockSpec((H, H), lambda i: (0, 0)),
        ],
        out_specs=[
            pl.BlockSpec((_BT, H), lambda i: (i, 0)),
            pl.BlockSpec((_BT, H), lambda i: (i, 0)),
        ],
        out_shape=[
            jax.ShapeDtypeStruct((NPAD, H), jnp.float32),
            jax.ShapeDtypeStruct((NPAD, H), jnp.float32),
        ],
    )(nf_p, wa, wb)


@functools.cache
def _sc_mesh():
    return plsc.VectorSubcoreMesh(
        core_axis_name="c", subcore_axis_name="s", num_cores=NC, num_subcores=NS)


@functools.cache
def _gather_kernel_build():
    buf_set = [
        pltpu.VMEM((CH, H), jnp.float32),
        pltpu.VMEM((CH, H), jnp.float32),
        pltpu.VMEM((6 * CH,), jnp.float32),
        pltpu.VMEM((3 * CH,), jnp.float32),
        pltpu.SemaphoreType.DMA,
        pltpu.SemaphoreType.DMA,
    ]
    return pl.kernel(
        _gather_body,
        out_type=[
            jax.ShapeDtypeStruct((E, H), jnp.float32),
            jax.ShapeDtypeStruct((3 * E,), jnp.float32),
        ],
        mesh=_sc_mesh(),
        scratch_types=[
            pltpu.VMEM((NCHUNK_PAD, CH), jnp.int32),
            pltpu.VMEM((NCHUNK_PAD, CH), jnp.int32),
        ] + buf_set + buf_set,
    )


def _gather_body(ta_hbm, tb_hbm, px_hbm, py_hbm, pz_hbm, ir_hbm, ic_hbm,
                 pre_hbm, aux_hbm,
                 ir_v, ic_v,
                 abuf0, bbuf0, pbuf0, xbuf0, isem0, osem0,
                 abuf1, bbuf1, pbuf1, xbuf1, isem1, osem1):
    c = lax.axis_index("c")
    s = lax.axis_index("s")
    w = c * NS + s
    base = w * EPW
    AB = (abuf0, abuf1)
    BB = (bbuf0, bbuf1)
    PB = (pbuf0, pbuf1)
    XB = (xbuf0, xbuf1)
    ISEM = (isem0, isem1)
    OSEM = (osem0, osem1)
    pltpu.sync_copy(ir_hbm.at[pl.ds(w * NCHUNK_PAD, NCHUNK_PAD)], ir_v)
    pltpu.sync_copy(ic_hbm.at[pl.ds(w * NCHUNK_PAD, NCHUNK_PAD)], ic_v)

    def issue_in(j, b):
        pltpu.async_copy(ta_hbm.at[ir_v.at[j]], AB[b], ISEM[b])
        pltpu.async_copy(tb_hbm.at[ic_v.at[j]], BB[b], ISEM[b])
        for k, (tab, iv) in enumerate(
                ((px_hbm, ir_v), (px_hbm, ic_v), (py_hbm, ir_v),
                 (py_hbm, ic_v), (pz_hbm, ir_v), (pz_hbm, ic_v))):
            pltpu.async_copy(tab.at[iv.at[j]],
                             PB[b].at[pl.ds(k * CH, CH)], ISEM[b])

    def wait_in(b):
        pltpu.make_async_copy(ta_hbm.at[pl.ds(0, CH)], AB[b], ISEM[b]).wait()
        pltpu.make_async_copy(tb_hbm.at[pl.ds(0, CH)], BB[b], ISEM[b]).wait()
        for k in range(6):
            pltpu.make_async_copy(px_hbm.at[pl.ds(0, CH)],
                                  PB[b].at[pl.ds(k * CH, CH)], ISEM[b]).wait()

    def compute(b, nrows):
        a, bb, p, x = AB[b], BB[b], PB[b], XB[b]

        @pl.loop(0, nrows)
        def _(i):
            for t in range(H // 16):
                sl = pl.ds(t * 16, 16)
                a[i, sl] = a[i, sl] + bb[i, sl]

        for g in range(nrows // 16):
            for k in range(3):
                x[pl.ds(k * CH + g * 16, 16)] = (
                    p[pl.ds(2 * k * CH + g * 16, 16)]
                    - p[pl.ds((2 * k + 1) * CH + g * 16, 16)])

    def issue_out(j, b):
        pltpu.async_copy(AB[b], pre_hbm.at[pl.ds(base + j * CH, CH)], OSEM[b])
        for k in range(3):
            pltpu.async_copy(XB[b].at[pl.ds(k * CH, CH)],
                             aux_hbm.at[pl.ds(k * E + base + j * CH, CH)],
                             OSEM[b])

    def wait_out(b):
        pltpu.make_async_copy(AB[b], pre_hbm.at[pl.ds(0, CH)], OSEM[b]).wait()
        for k in range(3):
            pltpu.make_async_copy(XB[b].at[pl.ds(k * CH, CH)],
                                  aux_hbm.at[pl.ds(0, CH)], OSEM[b]).wait()

    issue_in(0, 0)

    @pl.loop(0, NFULL // 2)
    def _(p):
        j0 = p * 2
        for b in range(2):
            j = j0 + b
            o = 1 - b
            if b == 0:
                @pl.when(p > 0)
                def _():
                    wait_out(o)
            else:
                wait_out(o)
            issue_in(j + 1, o)
            wait_in(b)
            compute(b, CH)
            issue_out(j, b)

    wait_in(0)
    compute(0, REM)
    pltpu.sync_copy(AB[0].at[pl.ds(0, REM)],
                    pre_hbm.at[pl.ds(base + NFULL * CH, REM)])
    for k in range(3):
        pltpu.sync_copy(XB[0].at[pl.ds(k * CH, REM)],
                        aux_hbm.at[pl.ds(k * E + base + NFULL * CH, REM)])
    wait_out(1)


def _edge_body(pre_ref, dp_ref, ea_ref, w1c_ref, w1d_ref, b1_ref, w2_ref,
               b2_ref, pw1_ref, pb1_ref, pw2_ref, pb2_ref,
               msg_ref, wp_ref):
    pre = pre_ref[...]
    dp = dp_ref[...]
    dist = jnp.sum(dp * dp, axis=0, keepdims=True)
    ec = jnp.dot(ea_ref[...], w1c_ref[...], preferred_element_type=jnp.float32)
    h = jax.nn.silu(pre + ec + dist.T * w1d_ref[...] + b1_ref[...])
    msg = jax.nn.silu(
        jnp.dot(h, w2_ref[...], preferred_element_type=jnp.float32) + b2_ref[...])
    p1 = jax.nn.silu(
        jnp.dot(msg, pw1_ref[...], preferred_element_type=jnp.float32) + pb1_ref[...])
    sc = jnp.dot(p1, pw2_ref[...], preferred_element_type=jnp.float32) + pb2_ref[...]
    msg_ref[...] = msg
    wp_ref[...] = dp * sc.T


_BE = 2560


def _edge_call(pre, dp, ea, w1c, w1d, b1, w2, b2, pw1, pb1, pw2, pb2):
    cmap = lambda i: (0, 0)
    return pl.pallas_call(
        _edge_body,
        grid=(E // _BE,),
        in_specs=[
            pl.BlockSpec((_BE, H), lambda i: (i, 0)),
            pl.BlockSpec((3, _BE), lambda i: (0, i)),
            pl.BlockSpec((_BE, 16), lambda i: (i, 0)),
            pl.BlockSpec((16, H), cmap),
            pl.BlockSpec((1, H), cmap),
            pl.BlockSpec((1, H), cmap),
            pl.BlockSpec((H, H), cmap),
            pl.BlockSpec((1, H), cmap),
            pl.BlockSpec((H, H), cmap),
            pl.BlockSpec((1, H), cmap),
            pl.BlockSpec((H, 1), cmap),
            pl.BlockSpec((1, 1), cmap),
        ],
        out_specs=[
            pl.BlockSpec((_BE, H), lambda i: (i, 0)),
            pl.BlockSpec((3, _BE), lambda i: (0, i)),
        ],
        out_shape=[
            jax.ShapeDtypeStruct((EPAD, H), jnp.float32),
            jax.ShapeDtypeStruct((3, EPAD), jnp.float32),
        ],
    )(pre, dp, ea, w1c, w1d, b1, w2, b2, pw1, pb1, pw2, pb2)


@functools.cache
def _scatter_kernel_build():
    buf_set = [
        pltpu.VMEM((CH, H), jnp.float32),
        pltpu.VMEM((3 * CH,), jnp.float32),
        pltpu.SemaphoreType.DMA,
        pltpu.SemaphoreType.DMA,
    ]
    return pl.kernel(
        _scatter_body,
        out_type=[
            jax.ShapeDtypeStruct((NC * NPAD, H), jnp.float32),
            jax.ShapeDtypeStruct((NC * 4 * NPAD,), jnp.float32),
        ],
        mesh=_sc_mesh(),
        scratch_types=[
            pltpu.VMEM((NCHUNK_PAD, CH), jnp.int32),
            pltpu.VMEM((CH,), jnp.float32),
            pltpu.VMEM((640,), jnp.float32),
            pltpu.VMEM_SHARED((NPAD, H), jnp.float32),
            pltpu.VMEM_SHARED((NPAD,), jnp.float32),
            pltpu.VMEM_SHARED((NPAD,), jnp.float32),
            pltpu.VMEM_SHARED((NPAD,), jnp.float32),
            pltpu.VMEM_SHARED((NPAD,), jnp.float32),
        ] + buf_set + buf_set,
    )


def _scatter_body(msg_hbm, wp_hbm, is_hbm, outm_hbm, outx_hbm,
                  is_v, ones, zbuf, accm, accx, accy, accz, accc,
                  bufm0, bw0, lsem0, asem0, bufm1, bw1, lsem1, asem1):
    c = lax.axis_index("c")
    s = lax.axis_index("s")
    w = c * NS + s
    base = w * EPW
    r0 = s * RPT
    nz = RPT // CH
    rz = RPT - nz * CH
    BM = (bufm0, bufm1)
    BW = (bw0, bw1)
    LSEM = (lsem0, lsem1)
    ASEM = (asem0, asem1)

    @pl.loop(0, CH)
    def _(i):
        for t in range(H // 16):
            bufm0[i, pl.ds(t * 16, 16)] = jnp.zeros((16,), jnp.float32)

    @pl.loop(0, 40)
    def _(i):
        zbuf[pl.ds(i * 16, 16)] = jnp.zeros((16,), jnp.float32)

    @pl.loop(0, CH // 16)
    def _(i):
        ones[pl.ds(i * 16, 16)] = jnp.full((16,), 1.0, jnp.float32)

    @pl.loop(0, nz)
    def _(k):
        pltpu.sync_copy(bufm0, accm.at[pl.ds(r0 + k * CH, CH)])

    pltpu.sync_copy(bufm0.at[pl.ds(0, rz)], accm.at[pl.ds(r0 + nz * CH, rz)])
    for acc in (accx, accy, accz, accc):
        pltpu.sync_copy(zbuf.at[pl.ds(0, RPT)], acc.at[pl.ds(r0, RPT)])
    plsc.subcore_barrier()

    pltpu.sync_copy(is_hbm.at[pl.ds(w * NCHUNK_PAD, NCHUNK_PAD)], is_v)

    def issue_load(j, b):
        pltpu.async_copy(msg_hbm.at[pl.ds(base + j * CH, CH)], BM[b], LSEM[b])
        for k in range(3):
            pltpu.async_copy(wp_hbm.at[pl.ds(k * EPAD + base + j * CH, CH)],
                             BW[b].at[pl.ds(k * CH, CH)], LSEM[b])

    def wait_load(b):
        pltpu.make_async_copy(msg_hbm.at[pl.ds(0, CH)], BM[b], LSEM[b]).wait()
        for k in range(3):
            pltpu.make_async_copy(wp_hbm.at[pl.ds(0, CH)],
                                  BW[b].at[pl.ds(k * CH, CH)], LSEM[b]).wait()

    def issue_adds(j, b):
        idx = is_v.at[j]
        pltpu.async_copy(BM[b], accm.at[idx], ASEM[b], add=True)
        pltpu.async_copy(BW[b].at[pl.ds(0, CH)], accx.at[idx], ASEM[b], add=True)
        pltpu.async_copy(BW[b].at[pl.ds(CH, CH)], accy.at[idx], ASEM[b], add=True)
        pltpu.async_copy(BW[b].at[pl.ds(2 * CH, CH)], accz.at[idx], ASEM[b],
                         add=True)
        pltpu.async_copy(ones, accc.at[idx], ASEM[b], add=True)

    def wait_adds(b):
        pltpu.make_async_copy(BM[b], accm.at[pl.ds(0, CH)], ASEM[b]).wait()
        for k in range(3):
            pltpu.make_async_copy(BW[b].at[pl.ds(k * CH, CH)],
                                  accx.at[pl.ds(0, CH)], ASEM[b]).wait()
        pltpu.make_async_copy(ones, accc.at[pl.ds(0, CH)], ASEM[b]).wait()

    issue_load(0, 0)

    @pl.loop(0, NCHUNK // 2)
    def _(p):
        j0 = p * 2
        for b in range(2):
            j = j0 + b
            o = 1 - b
            if b == 0:
                @pl.when(p > 0)
                def _():
                    wait_adds(o)
            else:
                wait_adds(o)
            issue_load(j + 1, o)
            wait_load(b)
            issue_adds(j, b)

    wait_load(0)
    issue_adds(NCHUNK - 1, 0)
    wait_adds(0)
    wait_adds(1)
    plsc.subcore_barrier()

    @pl.loop(0, nz)
    def _(k):
        pltpu.sync_copy(accm.at[pl.ds(r0 + k * CH, CH)], bufm0)
        pltpu.sync_copy(bufm0, outm_hbm.at[pl.ds(c * NPAD + r0 + k * CH, CH)])

    pltpu.sync_copy(accm.at[pl.ds(r0 + nz * CH, rz)], bufm0.at[pl.ds(0, rz)])
    pltpu.sync_copy(bufm0.at[pl.ds(0, rz)],
                    outm_hbm.at[pl.ds(c * NPAD + nz * CH + r0, rz)])
    for k, acc in enumerate((accx, accy, accz, accc)):
        pltpu.sync_copy(acc.at[pl.ds(r0, RPT)], zbuf.at[pl.ds(0, RPT)])
        pltpu.sync_copy(zbuf.at[pl.ds(0, RPT)],
                        outx_hbm.at[pl.ds((c * 4 + k) * NPAD + r0, RPT)])


def _node_body(accm_ref, accx_ref, nf_ref, pos_ref, vel_ref,
               nw1a_ref, nw1b_ref, nb1_ref, nw2_ref, nb2_ref,
               vw1_ref, vb1_ref, vw2_ref, vb2_ref,
               feat_out_ref, pos_out_ref):
    aux = accx_ref[0] + accx_ref[1]
    cnt = jnp.maximum(aux[:, 3:4], 1.0)
    magg = (accm_ref[0] + accm_ref[1]) / cnt
    pagg = aux[:, :3] / cnt
    nf = nf_ref[...]
    h = jax.nn.silu(
        jnp.dot(nf, nw1a_ref[...], preferred_element_type=jnp.float32)
        + jnp.dot(magg, nw1b_ref[...], preferred_element_type=jnp.float32)
        + nb1_ref[...])
    feat_out_ref[...] = (
        jnp.dot(h, nw2_ref[...], preferred_element_type=jnp.float32) + nb2_ref[...])
    v = jax.nn.silu(
        jnp.dot(nf, vw1_ref[...], preferred_element_type=jnp.float32) + vb1_ref[...])
    vs = jnp.dot(v, vw2_ref[...], preferred_element_type=jnp.float32) + vb2_ref[...]
    pos_out_ref[...] = pos_ref[...] + pagg + vs * vel_ref[...]


_BN = 2000


def _node_call(accm, accx, nf, pos, vel, nw1a, nw1b, nb1, nw2, nb2,
               vw1, vb1, vw2, vb2):
    cmap = lambda i: (0, 0)
    return pl.pallas_call(
        _node_body,
        grid=(N // _BN,),
        in_specs=[
            pl.BlockSpec((2, _BN, H), lambda i: (0, i, 0)),
            pl.BlockSpec((2, _BN, 4), lambda i: (0, i, 0)),
            pl.BlockSpec((_BN, H), lambda i: (i, 0)),
            pl.BlockSpec((_BN, 3), lambda i: (i, 0)),
            pl.BlockSpec((_BN, 3), lambda i: (i, 0)),
            pl.BlockSpec((H, H), cmap),
            pl.BlockSpec((H, H), cmap),
            pl.BlockSpec((1, H), cmap),
            pl.BlockSpec((H, H), cmap),
            pl.BlockSpec((1, H), cmap),
            pl.BlockSpec((H, H), cmap),
            pl.BlockSpec((1, H), cmap),
            pl.BlockSpec((H, 1), cmap),
            pl.BlockSpec((1, 1), cmap),
        ],
        out_specs=[
            pl.BlockSpec((_BN, H), lambda i: (i, 0)),
            pl.BlockSpec((_BN, 3), lambda i: (i, 0)),
        ],
        out_shape=[
            jax.ShapeDtypeStruct((N, H), jnp.float32),
            jax.ShapeDtypeStruct((N, 3), jnp.float32),
        ],
    )(accm, accx, nf, pos, vel, nw1a, nw1b, nb1, nw2, nb2, vw1, vb1, vw2, vb2)


def _pad_idx(ix):
    ix = ix.reshape(NW, EPW)
    pad = jnp.full((NW, NCHUNK_PAD * CH - EPW), N, jnp.int32)
    return jnp.concatenate([ix, pad], axis=1).reshape(NW * NCHUNK_PAD, CH)


def kernel(node_feat, node_pos, node_vel, edge_index, edge_attr,
           msg_W1, msg_b1, msg_W2, msg_b2,
           pos_W1, pos_b1, pos_W2, pos_b2,
           node_W1, node_b1, node_W2, node_b2,
           vel_W1, vel_b1, vel_W2, vel_b2):
    ir = _pad_idx(edge_index[0])
    ic = _pad_idx(edge_index[1])
    nf_p = jnp.pad(node_feat, ((0, NPAD - N), (0, 0)))
    pos_p = jnp.pad(node_pos, ((0, NPAD - N), (0, 0)))
    px, py, pz = pos_p[:, 0], pos_p[:, 1], pos_p[:, 2]

    ta, tb = _tables_call(nf_p, msg_W1[:H], msg_W1[H:2 * H])
    pre, dp = _gather_kernel_build()(ta, tb, px, py, pz, ir, ic)
    msg, wp = _edge_call(pre, dp.reshape(3, E), edge_attr,
                         msg_W1[2 * H:2 * H + 16], msg_W1[2 * H + 16:],
                         msg_b1.reshape(1, H), msg_W2, msg_b2.reshape(1, H),
                         pos_W1, pos_b1.reshape(1, H), pos_W2,
                         pos_b2.reshape(1, 1))
    accm, accx = _scatter_kernel_build()(msg, wp.reshape(3 * EPAD), ir)
    accm = accm.reshape(NC, NPAD, H)
    accx = jnp.transpose(accx.reshape(NC, 4, NPAD), (0, 2, 1))
    return _node_call(accm, accx, node_feat, node_pos, node_vel,
                      node_W1[:H], node_W1[H:], node_b1.reshape(1, H),
                      node_W2, node_b2.reshape(1, H),
                      vel_W1, vel_b1.reshape(1, H), vel_W2, vel_b2.reshape(1, 1))
